# baseline (speedup 1.0000x reference)
import functools

import jax
import jax.numpy as jnp
from jax import lax
from jax.experimental import pallas as pl
from jax.experimental.pallas import tpu as pltpu
from jax.experimental.pallas import tpu_sc as plsc

RT = 256
CT = 256
NEG = -1e30


def _flash_body(s_ref, xq, kh, vh, slr, slc3, out, acc, m, l,
                kbuf, vbuf, ksem, vsem, *, nct):
    r = pl.program_id(0)
    lo = s_ref[0, r]
    span = s_ref[1, r] - lo

    def kcopy(i, slot):
        return pltpu.make_async_copy(
            kh.at[pl.ds((lo + i) * CT, CT), :], kbuf.at[slot], ksem.at[slot])

    def vcopy(i, slot):
        return pltpu.make_async_copy(
            vh.at[pl.ds((lo + i) * CT, CT), :], vbuf.at[slot], vsem.at[slot])

    kcopy(0, 0).start()
    vcopy(0, 0).start()

    def body(i, _):
        slot = lax.rem(i, 2)

        @pl.when(i + 1 < span)
        def _prefetch():
            kcopy(i + 1, 1 - slot).start()
            vcopy(i + 1, 1 - slot).start()

        kcopy(i, slot).wait()
        vcopy(i, slot).wait()

        q = xq[...]
        k = kbuf[slot]
        logits = lax.dot_general(q, k, (((1,), (1,)), ((), ())),
                                 preferred_element_type=jnp.float32)
        slc = slc3[lo + i]
        mask = slr[...] == slc
        lm = jnp.where(mask, logits, NEG)
        m_old = jnp.max(m[...], axis=1, keepdims=True)
        m_new = jnp.maximum(m_old, jnp.max(lm, axis=1, keepdims=True))
        alpha = jnp.exp(m_old - m_new)
        p = jnp.where(mask, jnp.exp(logits - m_new), 0.0)
        pv = lax.dot_general(p, vbuf[slot], (((1,), (0,)), ((), ())),
                             preferred_element_type=jnp.float32)
        l_old = jnp.max(l[...], axis=1, keepdims=True)
        l_new = l_old * alpha + jnp.sum(p, axis=1, keepdims=True)
        m[...] = jnp.broadcast_to(m_new, m.shape)
        l[...] = jnp.broadcast_to(l_new, l.shape)

        @pl.when((i == 0) & (span > 1))
        def _first():
            acc[...] = pv

        @pl.when((i > 0) & (i < span - 1))
        def _mid():
            acc[...] = acc[...] * alpha + pv

        @pl.when((i == span - 1) & (span > 1))
        def _last():
            out[...] = (acc[...] * alpha + pv) * (1.0 / l_new)

        @pl.when((i == 0) & (span == 1))
        def _only():
            out[...] = pv * (1.0 / l_new)

        return 0

    m[...] = jnp.full_like(m, NEG)
    l[...] = jnp.zeros_like(l)
    lax.fori_loop(0, span, body, 0)


def _attention(xq, xk, xv, slab, s, *, interpret=False):
    n, d = xq.shape
    nrt, nct = n // RT, n // CT
    r_idx = lambda r, s_ref: (r, 0)
    grid_spec = pltpu.PrefetchScalarGridSpec(
        num_scalar_prefetch=1,
        grid=(nrt,),
        in_specs=[
            pl.BlockSpec((RT, d), r_idx),
            pl.BlockSpec(memory_space=pl.ANY),
            pl.BlockSpec(memory_space=pl.ANY),
            pl.BlockSpec((RT, 1), r_idx),
            pl.BlockSpec((nct, 1, CT), lambda r, s_ref: (0, 0, 0)),
        ],
        out_specs=pl.BlockSpec((RT, d), r_idx),
        scratch_shapes=[
            pltpu.VMEM((RT, d), jnp.float32),
            pltpu.VMEM((RT, 128), jnp.float32),
            pltpu.VMEM((RT, 128), jnp.float32),
            pltpu.VMEM((2, CT, d), jnp.float32),
            pltpu.VMEM((2, CT, d), jnp.float32),
            pltpu.SemaphoreType.DMA((2,)),
            pltpu.SemaphoreType.DMA((2,)),
        ],
    )
    fn = pl.pallas_call(
        functools.partial(_flash_body, nct=nct),
        grid_spec=grid_spec,
        out_shape=jax.ShapeDtypeStruct((n, d), jnp.float32),
        compiler_params=pltpu.CompilerParams(
            dimension_semantics=("arbitrary",)),
        interpret=interpret,
    )
    return fn(s, xq, xk, xv, slab.reshape(-1, 1), slab.reshape(nct, 1, CT))


def _gather_scale_rows(table, idx, w, b):
    return table[idx] * w[:, None] + b[:, None]


def _gather_rows(table, idx):
    return table[idx]


def kernel(x, labels, W, b):
    B, C, h, w = x.shape
    N = B * C
    D = h * w
    OC = 3 * C
    x2d = x.reshape(N, D)
    labels = labels.astype(jnp.int32)

    lab8 = jnp.arange(8, dtype=jnp.int32)
    counts = jnp.sum(labels[:, None] == lab8[None, :], axis=0)
    offs = jnp.concatenate([jnp.zeros((1,), jnp.int32),
                            jnp.cumsum(counts).astype(jnp.int32)])
    eq = (labels[:, None] == lab8[None, :]).astype(jnp.int32)
    rank = jnp.cumsum(eq, axis=0) - eq
    base = jnp.sum(eq * offs[None, :8], axis=1)
    pos = base + jnp.sum(eq * rank, axis=1)
    pos = pos.astype(jnp.int32)
    perm = jnp.zeros((N,), jnp.int32).at[pos].set(
        jnp.arange(N, dtype=jnp.int32), mode='drop')

    i_n = jnp.arange(N, dtype=jnp.int32)
    slab = (jnp.sum(i_n[:, None] >= offs[None, 1:], axis=1)).astype(jnp.int32)

    n_all = jnp.concatenate([perm, perm + N, perm + 2 * N])
    j_all = n_all % OC
    src = ((n_all // OC) * C + j_all // 3).astype(jnp.int32)

    xg = _gather_scale_rows(x2d, src, W[j_all], b[j_all])

    nrt = N // RT
    tlo = jnp.arange(nrt, dtype=jnp.int32) * RT
    thi = tlo + (RT - 1)
    start_row = jnp.max(jnp.where(offs[None, :] <= tlo[:, None],
                                  offs[None, :], 0), axis=1)
    glast = jnp.sum(thi[:, None] >= offs[None, 1:], axis=1)
    end_row = jnp.sum(jnp.where(jnp.arange(9)[None, :] == (glast + 1)[:, None],
                                offs[None, :], 0), axis=1)
    lo = start_row // CT
    hi = (end_row + CT - 1) // CT
    s = jnp.stack([lo, hi]).astype(jnp.int32)

    os_ = _attention(xg[:N], xg[N:2 * N], xg[2 * N:], slab, s)

    out = _gather_rows(os_, pos)
    return out[None]


# speedup vs baseline: 1.2449x; 1.2449x over previous
import functools

import jax
import jax.numpy as jnp
from jax import lax
from jax.experimental import pallas as pl
from jax.experimental.pallas import tpu as pltpu
from jax.experimental.pallas import tpu_sc as plsc

RT = 256
CT = 256
NEG = -1e30


def _flash_body(s_ref, xq, kh, vh, wqr, bqr, slr, wk3, bk3, wv3, bv3, slc3,
                out, acc, m, l, qs, sq, kbuf, vbuf, ksem, vsem, *, nct):
    r = pl.program_id(0)
    lo = s_ref[0, r]
    span = s_ref[1, r] - lo

    def kcopy(i, slot):
        return pltpu.make_async_copy(
            kh.at[pl.ds((lo + i) * CT, CT), :], kbuf.at[slot], ksem.at[slot])

    def vcopy(i, slot):
        return pltpu.make_async_copy(
            vh.at[pl.ds((lo + i) * CT, CT), :], vbuf.at[slot], vsem.at[slot])

    kcopy(0, 0).start()
    vcopy(0, 0).start()

    q = xq[...] * wqr[...] + bqr[...]
    qs[...] = q
    sq[...] = jnp.broadcast_to(jnp.sum(q, axis=1, keepdims=True), sq.shape)
    m[...] = jnp.full_like(m, NEG)
    l[...] = jnp.zeros_like(l)

    def body(i, _):
        slot = lax.rem(i, 2)

        @pl.when(i + 1 < span)
        def _prefetch():
            kcopy(i + 1, 1 - slot).start()
            vcopy(i + 1, 1 - slot).start()

        kcopy(i, slot).wait()
        vcopy(i, slot).wait()

        g = lax.dot_general(qs[...], kbuf[slot], (((1,), (1,)), ((), ())),
                            preferred_element_type=jnp.float32)
        wk = wk3[lo + i]
        bk = bk3[lo + i]
        sqv = jnp.max(sq[...], axis=1, keepdims=True)
        logits = g * wk + sqv * bk
        slc = slc3[lo + i]
        mask = slr[...] == slc
        lm = jnp.where(mask, logits, NEG)
        m_old = jnp.max(m[...], axis=1, keepdims=True)
        m_new = jnp.maximum(m_old, jnp.max(lm, axis=1, keepdims=True))
        alpha = jnp.exp(m_old - m_new)
        p = jnp.where(mask, jnp.exp(logits - m_new), 0.0)
        pw = p * wv3[lo + i]
        pv = lax.dot_general(pw, vbuf[slot], (((1,), (0,)), ((), ())),
                             preferred_element_type=jnp.float32)
        pb = jnp.sum(p * bv3[lo + i], axis=1, keepdims=True)
        l_old = jnp.max(l[...], axis=1, keepdims=True)
        l_new = l_old * alpha + jnp.sum(p, axis=1, keepdims=True)
        m[...] = jnp.broadcast_to(m_new, m.shape)
        l[...] = jnp.broadcast_to(l_new, l.shape)

        @pl.when((i == 0) & (span > 1))
        def _first():
            acc[...] = pv + pb

        @pl.when((i > 0) & (i < span - 1))
        def _mid():
            acc[...] = acc[...] * alpha + (pv + pb)

        @pl.when((i == span - 1) & (span > 1))
        def _last():
            out[...] = (acc[...] * alpha + (pv + pb)) * (1.0 / l_new)

        @pl.when((i == 0) & (span == 1))
        def _only():
            out[...] = (pv + pb) * (1.0 / l_new)

        return 0

    lax.fori_loop(0, span, body, 0)


def _attention(xq, xk, xv, w_all, b_all, slab, s, *, interpret=False):
    n, d = xq.shape
    nrt, nct = n // RT, n // CT
    r_idx = lambda r, s_ref: (r, 0)
    c3 = lambda r, s_ref: (0, 0, 0)
    grid_spec = pltpu.PrefetchScalarGridSpec(
        num_scalar_prefetch=1,
        grid=(nrt,),
        in_specs=[
            pl.BlockSpec((RT, d), r_idx),
            pl.BlockSpec(memory_space=pl.ANY),
            pl.BlockSpec(memory_space=pl.ANY),
            pl.BlockSpec((RT, 1), r_idx),
            pl.BlockSpec((RT, 1), r_idx),
            pl.BlockSpec((RT, 1), r_idx),
            pl.BlockSpec((nct, 1, CT), c3),
            pl.BlockSpec((nct, 1, CT), c3),
            pl.BlockSpec((nct, 1, CT), c3),
            pl.BlockSpec((nct, 1, CT), c3),
            pl.BlockSpec((nct, 1, CT), c3),
        ],
        out_specs=pl.BlockSpec((RT, d), r_idx),
        scratch_shapes=[
            pltpu.VMEM((RT, d), jnp.float32),
            pltpu.VMEM((RT, 128), jnp.float32),
            pltpu.VMEM((RT, 128), jnp.float32),
            pltpu.VMEM((RT, d), jnp.float32),
            pltpu.VMEM((RT, 128), jnp.float32),
            pltpu.VMEM((2, CT, d), jnp.float32),
            pltpu.VMEM((2, CT, d), jnp.float32),
            pltpu.SemaphoreType.DMA((2,)),
            pltpu.SemaphoreType.DMA((2,)),
        ],
    )
    fn = pl.pallas_call(
        functools.partial(_flash_body, nct=nct),
        grid_spec=grid_spec,
        out_shape=jax.ShapeDtypeStruct((n, d), jnp.float32),
        compiler_params=pltpu.CompilerParams(
            dimension_semantics=("arbitrary",)),
        interpret=interpret,
    )
    col = lambda a: a.reshape(-1, 1)
    row3 = lambda a: a.reshape(nct, 1, CT)
    return fn(s, xq, xk, xv,
              col(w_all[:n]), col(b_all[:n]), col(slab),
              row3(w_all[n:2 * n]), row3(b_all[n:2 * n]),
              row3(w_all[2 * n:]), row3(b_all[2 * n:]),
              slab.reshape(nct, 1, CT))


def _gather_scale_rows(table, idx, w, b):
    return table[idx] * w[:, None] + b[:, None]


def _gather_rows(table, idx):
    return table[idx]


def kernel(x, labels, W, b):
    B, C, h, w = x.shape
    N = B * C
    D = h * w
    OC = 3 * C
    x2d = x.reshape(N, D)
    labels = labels.astype(jnp.int32)

    lab8 = jnp.arange(8, dtype=jnp.int32)
    counts = jnp.sum(labels[:, None] == lab8[None, :], axis=0)
    offs = jnp.concatenate([jnp.zeros((1,), jnp.int32),
                            jnp.cumsum(counts).astype(jnp.int32)])
    eq = (labels[:, None] == lab8[None, :]).astype(jnp.int32)
    rank = jnp.cumsum(eq, axis=0) - eq
    base = jnp.sum(eq * offs[None, :8], axis=1)
    pos = base + jnp.sum(eq * rank, axis=1)
    pos = pos.astype(jnp.int32)
    perm = jnp.zeros((N,), jnp.int32).at[pos].set(
        jnp.arange(N, dtype=jnp.int32), mode='drop')

    i_n = jnp.arange(N, dtype=jnp.int32)
    slab = (jnp.sum(i_n[:, None] >= offs[None, 1:], axis=1)).astype(jnp.int32)

    n_all = jnp.concatenate([perm, perm + N, perm + 2 * N])
    j_all = n_all % OC
    src = ((n_all // OC) * C + j_all // 3).astype(jnp.int32)

    xg = _gather_rows(x2d, src)
    w_all = W[j_all]
    b_all = b[j_all]

    nrt = N // RT
    tlo = jnp.arange(nrt, dtype=jnp.int32) * RT
    thi = tlo + (RT - 1)
    start_row = jnp.max(jnp.where(offs[None, :] <= tlo[:, None],
                                  offs[None, :], 0), axis=1)
    glast = jnp.sum(thi[:, None] >= offs[None, 1:], axis=1)
    end_row = jnp.sum(jnp.where(jnp.arange(9)[None, :] == (glast + 1)[:, None],
                                offs[None, :], 0), axis=1)
    lo = start_row // CT
    hi = (end_row + CT - 1) // CT
    s = jnp.stack([lo, hi]).astype(jnp.int32)

    os_ = _attention(xg[:N], xg[N:2 * N], xg[2 * N:], w_all, b_all, slab, s)

    out = _gather_rows(os_, pos)
    return out[None]


# speedup vs baseline: 1.4829x; 1.1912x over previous
import functools

import jax
import jax.numpy as jnp
from jax import lax
from jax.experimental import pallas as pl
from jax.experimental.pallas import tpu as pltpu
from jax.experimental.pallas import tpu_sc as plsc

RT = 256
CT = 256
NEG = -1e30


def _flash_body(s_ref, xq, kh, vh, wqr, bqr, slr, wk3, bk3, wv3, bv3, slc3,
                out, acc, m, l, qs, sq, kbuf, vbuf, ksem, vsem, *, nct):
    r = pl.program_id(0)
    lo = s_ref[0, r]
    span = s_ref[1, r] - lo

    def kcopy(i, slot):
        return pltpu.make_async_copy(
            kh.at[pl.ds((lo + i) * CT, CT), :], kbuf.at[slot], ksem.at[slot])

    def vcopy(i, slot):
        return pltpu.make_async_copy(
            vh.at[pl.ds((lo + i) * CT, CT), :], vbuf.at[slot], vsem.at[slot])

    kcopy(0, 0).start()
    vcopy(0, 0).start()

    q = xq[...] * wqr[...] + bqr[...]
    qs[...] = q
    sq[...] = jnp.broadcast_to(jnp.sum(q, axis=1, keepdims=True), sq.shape)
    m[...] = jnp.full_like(m, NEG)
    l[...] = jnp.zeros_like(l)

    def body(i, _):
        slot = lax.rem(i, 2)

        @pl.when(i + 1 < span)
        def _prefetch():
            kcopy(i + 1, 1 - slot).start()
            vcopy(i + 1, 1 - slot).start()

        kcopy(i, slot).wait()
        vcopy(i, slot).wait()

        g = lax.dot_general(qs[...], kbuf[slot], (((1,), (1,)), ((), ())),
                            preferred_element_type=jnp.float32)
        wk = wk3[lo + i]
        bk = bk3[lo + i]
        sqv = jnp.max(sq[...], axis=1, keepdims=True)
        logits = g * wk + sqv * bk
        slc = slc3[lo + i]
        mask = slr[...] == slc
        lm = jnp.where(mask, logits, NEG)
        m_old = jnp.max(m[...], axis=1, keepdims=True)
        m_new = jnp.maximum(m_old, jnp.max(lm, axis=1, keepdims=True))
        alpha = jnp.exp(m_old - m_new)
        p = jnp.where(mask, jnp.exp(logits - m_new), 0.0)
        pw = p * wv3[lo + i]
        pv = lax.dot_general(pw, vbuf[slot], (((1,), (0,)), ((), ())),
                             preferred_element_type=jnp.float32)
        pb = jnp.sum(p * bv3[lo + i], axis=1, keepdims=True)
        l_old = jnp.max(l[...], axis=1, keepdims=True)
        l_new = l_old * alpha + jnp.sum(p, axis=1, keepdims=True)
        m[...] = jnp.broadcast_to(m_new, m.shape)
        l[...] = jnp.broadcast_to(l_new, l.shape)

        @pl.when((i == 0) & (span > 1))
        def _first():
            acc[...] = pv + pb

        @pl.when((i > 0) & (i < span - 1))
        def _mid():
            acc[...] = acc[...] * alpha + (pv + pb)

        @pl.when((i == span - 1) & (span > 1))
        def _last():
            out[...] = (acc[...] * alpha + (pv + pb)) * (1.0 / l_new)

        @pl.when((i == 0) & (span == 1))
        def _only():
            out[...] = (pv + pb) * (1.0 / l_new)

        return 0

    lax.fori_loop(0, span, body, 0)


def _attention(xq, xk, xv, w_all, b_all, slab, s, *, interpret=False):
    n, d = xq.shape
    nrt, nct = n // RT, n // CT
    r_idx = lambda r, s_ref: (r, 0)
    c3 = lambda r, s_ref: (0, 0, 0)
    grid_spec = pltpu.PrefetchScalarGridSpec(
        num_scalar_prefetch=1,
        grid=(nrt,),
        in_specs=[
            pl.BlockSpec((RT, d), r_idx),
            pl.BlockSpec(memory_space=pl.ANY),
            pl.BlockSpec(memory_space=pl.ANY),
            pl.BlockSpec((RT, 1), r_idx),
            pl.BlockSpec((RT, 1), r_idx),
            pl.BlockSpec((RT, 1), r_idx),
            pl.BlockSpec((nct, 1, CT), c3),
            pl.BlockSpec((nct, 1, CT), c3),
            pl.BlockSpec((nct, 1, CT), c3),
            pl.BlockSpec((nct, 1, CT), c3),
            pl.BlockSpec((nct, 1, CT), c3),
        ],
        out_specs=pl.BlockSpec((RT, d), r_idx),
        scratch_shapes=[
            pltpu.VMEM((RT, d), jnp.float32),
            pltpu.VMEM((RT, 128), jnp.float32),
            pltpu.VMEM((RT, 128), jnp.float32),
            pltpu.VMEM((RT, d), jnp.float32),
            pltpu.VMEM((RT, 128), jnp.float32),
            pltpu.VMEM((2, CT, d), jnp.float32),
            pltpu.VMEM((2, CT, d), jnp.float32),
            pltpu.SemaphoreType.DMA((2,)),
            pltpu.SemaphoreType.DMA((2,)),
        ],
    )
    fn = pl.pallas_call(
        functools.partial(_flash_body, nct=nct),
        grid_spec=grid_spec,
        out_shape=jax.ShapeDtypeStruct((n, d), jnp.float32),
        compiler_params=pltpu.CompilerParams(
            dimension_semantics=("arbitrary",)),
        interpret=interpret,
    )
    col = lambda a: a.reshape(-1, 1)
    row3 = lambda a: a.reshape(nct, 1, CT)
    return fn(s, xq, xk, xv,
              col(w_all[:n]), col(b_all[:n]), col(slab),
              row3(w_all[n:2 * n]), row3(b_all[n:2 * n]),
              row3(w_all[2 * n:]), row3(b_all[2 * n:]),
              slab.reshape(nct, 1, CT))


def _gather_rows(table, idx):
    B = idx.shape[0]
    V, D = table.shape
    info = plsc.get_sparse_core_info()
    ncores, nsub = info.num_cores, info.num_subcores
    nw = ncores * nsub
    bpw = B // nw
    CH = 8
    nch = bpw // CH
    mesh = plsc.VectorSubcoreMesh(core_axis_name="c", subcore_axis_name="s")

    @functools.partial(
        pl.kernel, mesh=mesh,
        out_type=jax.ShapeDtypeStruct((B, D), jnp.float32),
        scratch_types=[
            pltpu.VMEM((bpw,), jnp.int32),
            pltpu.VMEM((2, CH, D), jnp.float32),
            pltpu.SemaphoreType.DMA,
        ],
    )
    def gk(table_hbm, idx_hbm, out_hbm, idx_v, rows_v, sem):
        wid = lax.axis_index("s") * ncores + lax.axis_index("c")
        base = wid * bpw
        pltpu.sync_copy(idx_hbm.at[pl.ds(base, bpw)], idx_v)

        def gather(i, slot):
            return pltpu.async_copy(
                table_hbm.at[idx_v.at[pl.ds(i * CH, CH)]], rows_v.at[slot], sem)

        g0 = gather(0, 0)

        def chunk(i, _):
            slot = lax.rem(i, 2)
            pltpu.make_async_copy(
                table_hbm.at[idx_v.at[pl.ds(i * CH, CH)]], rows_v.at[slot],
                sem).wait()

            @pl.when(i + 1 < nch)
            def _pref():
                gather(i + 1, 1 - slot)

            pltpu.sync_copy(rows_v.at[slot],
                            out_hbm.at[pl.ds(base + i * CH, CH)])
            return 0

        lax.fori_loop(0, nch, chunk, 0)

    return gk(table, idx)


def kernel(x, labels, W, b):
    B, C, h, w = x.shape
    N = B * C
    D = h * w
    OC = 3 * C
    x2d = x.reshape(N, D)
    labels = labels.astype(jnp.int32)

    lab8 = jnp.arange(8, dtype=jnp.int32)
    counts = jnp.sum(labels[:, None] == lab8[None, :], axis=0)
    offs = jnp.concatenate([jnp.zeros((1,), jnp.int32),
                            jnp.cumsum(counts).astype(jnp.int32)])
    eq = (labels[:, None] == lab8[None, :]).astype(jnp.int32)
    rank = jnp.cumsum(eq, axis=0) - eq
    base = jnp.sum(eq * offs[None, :8], axis=1)
    pos = base + jnp.sum(eq * rank, axis=1)
    pos = pos.astype(jnp.int32)
    perm = jnp.zeros((N,), jnp.int32).at[pos].set(
        jnp.arange(N, dtype=jnp.int32), mode='drop')

    i_n = jnp.arange(N, dtype=jnp.int32)
    slab = (jnp.sum(i_n[:, None] >= offs[None, 1:], axis=1)).astype(jnp.int32)

    n_all = jnp.concatenate([perm, perm + N, perm + 2 * N])
    j_all = n_all % OC
    src = ((n_all // OC) * C + j_all // 3).astype(jnp.int32)

    xg = _gather_rows(x2d, src)
    w_all = W[j_all]
    b_all = b[j_all]

    nrt = N // RT
    tlo = jnp.arange(nrt, dtype=jnp.int32) * RT
    thi = tlo + (RT - 1)
    start_row = jnp.max(jnp.where(offs[None, :] <= tlo[:, None],
                                  offs[None, :], 0), axis=1)
    glast = jnp.sum(thi[:, None] >= offs[None, 1:], axis=1)
    end_row = jnp.sum(jnp.where(jnp.arange(9)[None, :] == (glast + 1)[:, None],
                                offs[None, :], 0), axis=1)
    lo = start_row // CT
    hi = (end_row + CT - 1) // CT
    s = jnp.stack([lo, hi]).astype(jnp.int32)

    os_ = _attention(xg[:N], xg[N:2 * N], xg[2 * N:], w_all, b_all, slab, s)

    out = _gather_rows(os_, pos)
    return out[None]


# speedup vs baseline: 1.7772x; 1.1984x over previous
import functools

import jax
import jax.numpy as jnp
from jax import lax
from jax.experimental import pallas as pl
from jax.experimental.pallas import tpu as pltpu
from jax.experimental.pallas import tpu_sc as plsc

RT = 256
CT = 256
NEG = -1e30


def _flash_body(s_ref, xq, kh, vh, wqr, bqr, slr, wk3, bk3, wv3, bv3, slc3,
                out, acc, m, l, qs, sq, kbuf, vbuf, ksem, vsem, *, nct):
    r = pl.program_id(0)
    lo = s_ref[0, r]
    span = s_ref[1, r] - lo

    def kcopy(i, slot):
        return pltpu.make_async_copy(
            kh.at[pl.ds((lo + i) * CT, CT), :], kbuf.at[slot], ksem.at[slot])

    def vcopy(i, slot):
        return pltpu.make_async_copy(
            vh.at[pl.ds((lo + i) * CT, CT), :], vbuf.at[slot], vsem.at[slot])

    kcopy(0, 0).start()
    vcopy(0, 0).start()

    q = xq[...] * wqr[...] + bqr[...]
    qs[...] = q
    sq[...] = jnp.broadcast_to(jnp.sum(q, axis=1, keepdims=True), sq.shape)
    m[...] = jnp.full_like(m, NEG)
    l[...] = jnp.zeros_like(l)

    def body(i, _):
        slot = lax.rem(i, 2)

        @pl.when(i + 1 < span)
        def _prefetch():
            kcopy(i + 1, 1 - slot).start()
            vcopy(i + 1, 1 - slot).start()

        kcopy(i, slot).wait()
        vcopy(i, slot).wait()

        g = lax.dot_general(qs[...], kbuf[slot], (((1,), (1,)), ((), ())),
                            preferred_element_type=jnp.float32)
        wk = wk3[lo + i]
        bk = bk3[lo + i]
        sqv = jnp.max(sq[...], axis=1, keepdims=True)
        logits = g * wk + sqv * bk
        slc = slc3[lo + i]
        mask = slr[...] == slc
        lm = jnp.where(mask, logits, NEG)
        m_old = jnp.max(m[...], axis=1, keepdims=True)
        m_new = jnp.maximum(m_old, jnp.max(lm, axis=1, keepdims=True))
        alpha = jnp.exp(m_old - m_new)
        p = jnp.where(mask, jnp.exp(logits - m_new), 0.0)
        pw = p * wv3[lo + i]
        pv = lax.dot_general(pw, vbuf[slot], (((1,), (0,)), ((), ())),
                             preferred_element_type=jnp.float32)
        pb = jnp.sum(p * bv3[lo + i], axis=1, keepdims=True)
        l_old = jnp.max(l[...], axis=1, keepdims=True)
        l_new = l_old * alpha + jnp.sum(p, axis=1, keepdims=True)
        m[...] = jnp.broadcast_to(m_new, m.shape)
        l[...] = jnp.broadcast_to(l_new, l.shape)

        @pl.when((i == 0) & (span > 1))
        def _first():
            acc[...] = pv + pb

        @pl.when((i > 0) & (i < span - 1))
        def _mid():
            acc[...] = acc[...] * alpha + (pv + pb)

        @pl.when((i == span - 1) & (span > 1))
        def _last():
            out[...] = (acc[...] * alpha + (pv + pb)) * (1.0 / l_new)

        @pl.when((i == 0) & (span == 1))
        def _only():
            out[...] = (pv + pb) * (1.0 / l_new)

        return 0

    lax.fori_loop(0, span, body, 0)


def _attention(xq, xk, xv, w_all, b_all, slab, s, *, interpret=False):
    n, d = xq.shape
    nrt, nct = n // RT, n // CT
    r_idx = lambda r, s_ref: (r, 0)
    c3 = lambda r, s_ref: (0, 0, 0)
    grid_spec = pltpu.PrefetchScalarGridSpec(
        num_scalar_prefetch=1,
        grid=(nrt,),
        in_specs=[
            pl.BlockSpec((RT, d), r_idx),
            pl.BlockSpec(memory_space=pl.ANY),
            pl.BlockSpec(memory_space=pl.ANY),
            pl.BlockSpec((RT, 1), r_idx),
            pl.BlockSpec((RT, 1), r_idx),
            pl.BlockSpec((RT, 1), r_idx),
            pl.BlockSpec((nct, 1, CT), c3),
            pl.BlockSpec((nct, 1, CT), c3),
            pl.BlockSpec((nct, 1, CT), c3),
            pl.BlockSpec((nct, 1, CT), c3),
            pl.BlockSpec((nct, 1, CT), c3),
        ],
        out_specs=pl.BlockSpec((RT, d), r_idx),
        scratch_shapes=[
            pltpu.VMEM((RT, d), jnp.float32),
            pltpu.VMEM((RT, 128), jnp.float32),
            pltpu.VMEM((RT, 128), jnp.float32),
            pltpu.VMEM((RT, d), jnp.float32),
            pltpu.VMEM((RT, 128), jnp.float32),
            pltpu.VMEM((2, CT, d), jnp.float32),
            pltpu.VMEM((2, CT, d), jnp.float32),
            pltpu.SemaphoreType.DMA((2,)),
            pltpu.SemaphoreType.DMA((2,)),
        ],
    )
    fn = pl.pallas_call(
        functools.partial(_flash_body, nct=nct),
        grid_spec=grid_spec,
        out_shape=jax.ShapeDtypeStruct((n, d), jnp.float32),
        compiler_params=pltpu.CompilerParams(
            dimension_semantics=("arbitrary",)),
        interpret=interpret,
    )
    col = lambda a: a.reshape(-1, 1)
    row3 = lambda a: a.reshape(nct, 1, CT)
    return fn(s, xq, xk, xv,
              col(w_all[:n]), col(b_all[:n]), col(slab),
              row3(w_all[n:2 * n]), row3(b_all[n:2 * n]),
              row3(w_all[2 * n:]), row3(b_all[2 * n:]),
              slab.reshape(nct, 1, CT))


def _gather_rows_multi(table, idx, nout):
    B = idx.shape[0] // nout
    V, D = table.shape
    info = plsc.get_sparse_core_info()
    ncores, nsub = info.num_cores, info.num_subcores
    nw = ncores * nsub
    bpw = B // nw
    CH = 8
    nch = bpw // CH
    mesh = plsc.VectorSubcoreMesh(core_axis_name="c", subcore_axis_name="s")

    @functools.partial(
        pl.kernel, mesh=mesh,
        out_type=[jax.ShapeDtypeStruct((B, D), jnp.float32)
                  for _ in range(nout)],
        scratch_types=[
            pltpu.VMEM((nout * bpw,), jnp.int32),
            pltpu.VMEM((2, CH, D), jnp.float32),
            pltpu.SemaphoreType.DMA,
        ],
    )
    def gk(table_hbm, idx_hbm, *rest):
        outs, idx_v, rows_v, sem = rest[:nout], rest[nout], rest[nout + 1], rest[nout + 2]
        wid = lax.axis_index("s") * ncores + lax.axis_index("c")
        base = wid * bpw
        for o in range(nout):
            pltpu.sync_copy(idx_hbm.at[pl.ds(o * B + base, bpw)],
                            idx_v.at[pl.ds(o * bpw, bpw)])

        def gather(j, slot):
            return pltpu.async_copy(
                table_hbm.at[idx_v.at[pl.ds(j * CH, CH)]], rows_v.at[slot], sem)

        gather(0, 0)

        for o, out_hbm in enumerate(outs):
            def chunk(i, _, o=o, out_hbm=out_hbm):
                j = o * nch + i
                slot = lax.rem(j, 2)
                pltpu.make_async_copy(
                    table_hbm.at[idx_v.at[pl.ds(j * CH, CH)]], rows_v.at[slot],
                    sem).wait()

                @pl.when(j + 1 < nout * nch)
                def _pref():
                    gather(j + 1, 1 - slot)

                pltpu.sync_copy(rows_v.at[slot],
                                out_hbm.at[pl.ds(base + i * CH, CH)])
                return 0

            lax.fori_loop(0, nch, chunk, 0)

    return gk(table, idx)


def _gather_rows(table, idx):
    return _gather_rows_multi(table, idx, 1)[0]


def kernel(x, labels, W, b):
    B, C, h, w = x.shape
    N = B * C
    D = h * w
    OC = 3 * C
    x2d = x.reshape(N, D)
    labels = labels.astype(jnp.int32)

    lab8 = jnp.arange(8, dtype=jnp.int32)
    counts = jnp.sum(labels[:, None] == lab8[None, :], axis=0)
    offs = jnp.concatenate([jnp.zeros((1,), jnp.int32),
                            jnp.cumsum(counts).astype(jnp.int32)])
    eq = (labels[:, None] == lab8[None, :]).astype(jnp.int32)
    rank = jnp.cumsum(eq, axis=0) - eq
    base = jnp.sum(eq * offs[None, :8], axis=1)
    pos = base + jnp.sum(eq * rank, axis=1)
    pos = pos.astype(jnp.int32)
    perm = jnp.zeros((N,), jnp.int32).at[pos].set(
        jnp.arange(N, dtype=jnp.int32), mode='drop')

    i_n = jnp.arange(N, dtype=jnp.int32)
    slab = (jnp.sum(i_n[:, None] >= offs[None, 1:], axis=1)).astype(jnp.int32)

    n_all = jnp.concatenate([perm, perm + N, perm + 2 * N])
    j_all = n_all % OC
    src = ((n_all // OC) * C + j_all // 3).astype(jnp.int32)

    xq, xk, xv = _gather_rows_multi(x2d, src, 3)
    w_all = W[j_all]
    b_all = b[j_all]

    nrt = N // RT
    tlo = jnp.arange(nrt, dtype=jnp.int32) * RT
    thi = tlo + (RT - 1)
    start_row = jnp.max(jnp.where(offs[None, :] <= tlo[:, None],
                                  offs[None, :], 0), axis=1)
    glast = jnp.sum(thi[:, None] >= offs[None, 1:], axis=1)
    end_row = jnp.sum(jnp.where(jnp.arange(9)[None, :] == (glast + 1)[:, None],
                                offs[None, :], 0), axis=1)
    lo = start_row // CT
    hi = (end_row + CT - 1) // CT
    s = jnp.stack([lo, hi]).astype(jnp.int32)

    os_ = _attention(xq, xk, xv, w_all, b_all, slab, s)

    out = _gather_rows(os_, pos)
    return out[None]
